# baseline (speedup 1.0000x reference)
import functools

import jax
import jax.numpy as jnp
from jax import lax
from jax.experimental import pallas as pl
from jax.experimental.pallas import tpu as pltpu
from jax.experimental.pallas import tpu_sc as plsc

_C = 128
_NT = 16
_BN = 1000
G_PAD = 8


def _sc_hist(row2d, ones_b, zeros_e, NP, E):
    nch = E // _C
    halfc = -(-nch // (2 * _NT)) * _NT
    cpt = halfc // _NT
    rpt = NP // _NT
    mesh = plsc.VectorSubcoreMesh(core_axis_name="c", subcore_axis_name="s")

    @functools.partial(
        pl.kernel, mesh=mesh,
        out_type=jax.ShapeDtypeStruct((2, NP, 16), jnp.float32),
        scratch_types=[
            pltpu.VMEM((cpt, _C), jnp.int32),
            pltpu.VMEM((_C, 16), jnp.float32),
            pltpu.VMEM_SHARED((NP, 16), jnp.float32),
        ],
        compiler_params=pltpu.CompilerParams(use_tc_tiling_on_sc=False),
    )
    def hist(row2d_hbm, ones_hbm, zeros_hbm, out_hbm, ridx_all, ones_v,
             acc_sh):
        c = lax.axis_index("c")
        s = lax.axis_index("s")
        off = c * halfc + s * cpt
        count = jnp.clip(nch - off, 0, cpt)
        pltpu.sync_copy(zeros_hbm, acc_sh.at[pl.ds(s * rpt, rpt)])
        pltpu.sync_copy(ones_hbm, ones_v)
        pltpu.sync_copy(row2d_hbm.at[pl.ds(off, cpt)], ridx_all)
        plsc.subcore_barrier()

        def body(j, carry):
            @pl.when(j < count)
            def _():
                pltpu.sync_copy(ones_v, acc_sh.at[ridx_all.at[j]], add=True)

            return carry

        lax.fori_loop(0, cpt, body, 0)
        plsc.subcore_barrier()
        pltpu.sync_copy(acc_sh.at[pl.ds(s * rpt, rpt)],
                        out_hbm.at[c, pl.ds(s * rpt, rpt)])

    return hist(row2d, ones_b, zeros_e)


def _sc_main(hs_cat, row23, col2d, zeros_m, N, NP, E):
    nch = E // _C
    cpt = -(-nch // (_NT * 8)) * 8
    G = 8
    ngrp = cpt // G
    rpt = NP // _NT
    mesh = plsc.VectorSubcoreMesh(core_axis_name="c", subcore_axis_name="s")

    @functools.partial(
        pl.kernel, mesh=mesh,
        out_type=jax.ShapeDtypeStruct((2, NP, 128), jnp.float32),
        scratch_types=[
            pltpu.VMEM((G, _C), jnp.int32),
            pltpu.VMEM((G, _C), jnp.int32),
            pltpu.VMEM((2, _C, 128), jnp.float32),
            pltpu.VMEM_SHARED((NP, 128), jnp.float32),
            pltpu.SemaphoreType.DMA,
            pltpu.SemaphoreType.DMA,
        ],
    )
    def main(hs_hbm, row23_hbm, col2d_hbm, zm_hbm,
             g0_hbm, ridx_g, cidx_g, gbuf, accm, sem0, sem1):
        c = lax.axis_index("c")
        s = lax.axis_index("s")
        off = s * cpt
        count = jnp.clip(nch - s * cpt, 0, cpt)
        pltpu.sync_copy(zm_hbm, accm.at[pl.ds(s * rpt, rpt)])
        plsc.subcore_barrier()
        sems = [sem0, sem1]

        for g in range(ngrp):
            pltpu.sync_copy(row23_hbm.at[c, pl.ds(off + g * G, G)], ridx_g)
            pltpu.sync_copy(col2d_hbm.at[pl.ds(off + g * G, G)], cidx_g)
            for j in range(G):
                k = g * G + j
                b = j % 2

                @pl.when(k < count)
                def _(j=j, b=b):
                    pltpu.async_copy(hs_hbm.at[ridx_g.at[j]], gbuf.at[b],
                                     sems[b])

                if j > 0:
                    kp = g * G + j - 1

                    @pl.when(kp < count)
                    def _(j=j, b=b):
                        pltpu.make_async_copy(hs_hbm.at[ridx_g.at[j - 1]],
                                              gbuf.at[1 - b],
                                              sems[1 - b]).wait()
                        pltpu.sync_copy(gbuf.at[1 - b],
                                        accm.at[cidx_g.at[j - 1]], add=True)

            kl = g * G + G - 1

            @pl.when(kl < count)
            def _():
                pltpu.make_async_copy(hs_hbm.at[ridx_g.at[G - 1]],
                                      gbuf.at[(G - 1) % 2],
                                      sems[(G - 1) % 2]).wait()
                pltpu.sync_copy(gbuf.at[(G - 1) % 2],
                                accm.at[cidx_g.at[G - 1]], add=True)

        plsc.subcore_barrier()
        pltpu.sync_copy(accm.at[pl.ds(s * rpt, rpt)],
                        g0_hbm.at[c, pl.ds(s * rpt, rpt)])

    return main(hs_cat, row23, col2d, zeros_m)


def _sc_ea(row2d, col2d, ea_flat, dinv, zeros_e, N, NP, E):
    nch = E // _C
    halfc = -(-nch // (2 * _NT)) * _NT
    cpt = halfc // _NT
    rpt = NP // _NT
    mesh = plsc.VectorSubcoreMesh(core_axis_name="c", subcore_axis_name="s")

    @functools.partial(
        pl.kernel, mesh=mesh,
        out_type=jax.ShapeDtypeStruct((2, NP, 16), jnp.float32),
        scratch_types=[
            pltpu.VMEM((cpt, _C), jnp.int32),
            pltpu.VMEM((cpt, _C), jnp.int32),
            pltpu.VMEM((cpt * _C * 16,), jnp.float32),
            pltpu.VMEM((_C, 16), jnp.float32),
            pltpu.VMEM((N,), jnp.float32),
            pltpu.VMEM_SHARED((NP, 16), jnp.float32),
        ],
        compiler_params=pltpu.CompilerParams(needs_layout_passes=False,
                                             use_tc_tiling_on_sc=False),
    )
    def eak(row2d_hbm, col2d_hbm, ea_hbm, dinv_hbm, ze_hbm,
            eap_hbm, ridx_all, cidx_all, ea_all, ea2, dinv_v, acce):
        c = lax.axis_index("c")
        s = lax.axis_index("s")
        off = c * halfc + s * cpt
        count = jnp.clip(nch - off, 0, cpt)
        pltpu.sync_copy(ze_hbm, acce.at[pl.ds(s * rpt, rpt)])
        pltpu.sync_copy(dinv_hbm, dinv_v)
        pltpu.sync_copy(row2d_hbm.at[pl.ds(off, cpt)], ridx_all)
        pltpu.sync_copy(col2d_hbm.at[pl.ds(off, cpt)], cidx_all)
        pltpu.sync_copy(ea_hbm.at[pl.ds(off * _C * 16, cpt * _C * 16)], ea_all)
        plsc.subcore_barrier()

        def ebody(j, carry):
            @pl.when(j < count)
            def _():
                ebase = j * (_C * 16)
                for q in range(_C // 16):
                    r16 = ridx_all[j, pl.ds(q * 16, 16)]
                    g16 = plsc.load_gather(dinv_v, [r16])
                    e16 = lax.broadcasted_iota(jnp.int32, (16,), 0) + (q * 16)
                    for ch in range(16):
                        chv = jnp.full((16,), ch, jnp.int32)
                        v = plsc.load_gather(ea_all, [ebase + e16 * 16 + chv])
                        plsc.store_scatter(ea2, [e16, chv], v * g16)
                pltpu.sync_copy(ea2, acce.at[cidx_all.at[j]], add=True)

            return carry

        lax.fori_loop(0, cpt, ebody, 0)
        plsc.subcore_barrier()
        pltpu.sync_copy(acce.at[pl.ds(s * rpt, rpt)],
                        eap_hbm.at[c, pl.ds(s * rpt, rpt)])

    return eak(row2d, col2d, ea_flat, dinv, zeros_e)


def _prep_body(a_ref, x_ref, dp_ref, hs_ref, dinv_ref):
    a = a_ref[0]
    x = x_ref[...]
    dp = dp_ref[...]
    cnt = dp[0, :, 0:1] + dp[1, :, 0:1]
    dinv = lax.rsqrt(cnt + 1.0)
    hidden = jnp.where(x >= 0, x, a * x)
    hs = hidden * dinv
    dinv_ref[...] = dinv
    hs_ref[0] = hs[:, :128]
    hs_ref[1] = hs[:, 128:]


def _tc_prep(prelu_a, x, deg_part):
    N, D = x.shape
    grid = (N // _BN,)
    return pl.pallas_call(
        _prep_body,
        grid=grid,
        in_specs=[pl.BlockSpec(memory_space=pltpu.SMEM),
                  pl.BlockSpec((_BN, D), lambda i: (i, 0)),
                  pl.BlockSpec((2, _BN, 16), lambda i: (0, i, 0))],
        out_specs=[pl.BlockSpec((2, _BN, 128), lambda i: (0, i, 0)),
                   pl.BlockSpec((_BN, 1), lambda i: (i, 0))],
        out_shape=[jax.ShapeDtypeStruct((2, N, 128), jnp.float32),
                   jax.ShapeDtypeStruct((N, 1), jnp.float32)],
    )(jnp.reshape(prelu_a, (1,)), x, deg_part)


def _final_body(g02_ref, eap_ref, dinv_ref, hs2_ref, gn_ref,
                wmt_ref, wst_ref, mm_ref, ms_ref, out_ref):
    dinv = dinv_ref[...]
    g0 = jnp.concatenate([g02_ref[0], g02_ref[1]], axis=1)
    hs = jnp.concatenate([hs2_ref[0], hs2_ref[1]], axis=1)
    ea = eap_ref[0] + eap_ref[1]
    ch = lax.broadcasted_iota(jnp.int32, (1, 16), 1)
    sl = ((ch == 7) | (ch == 9)).astype(jnp.float32)
    a16 = (ea + dinv * sl) * dinv
    h_full = (g0 + hs) * dinv
    mean = (jnp.dot(h_full, wmt_ref[...], preferred_element_type=jnp.float32)
            + jnp.dot(a16, mm_ref[...], preferred_element_type=jnp.float32))
    logstd = (jnp.dot(h_full, wst_ref[...], preferred_element_type=jnp.float32)
              + jnp.dot(a16, ms_ref[...], preferred_element_type=jnp.float32))
    out_ref[...] = gn_ref[...] * jnp.exp(logstd) + mean


def _tc_final(g02, eap, dinv2, hs2, gn, wmt, wst, mm, ms):
    N, D = gn.shape
    grid = (N // _BN,)
    return pl.pallas_call(
        _final_body,
        grid=grid,
        in_specs=[pl.BlockSpec((2, _BN, 128), lambda i: (0, i, 0)),
                  pl.BlockSpec((2, _BN, 16), lambda i: (0, i, 0)),
                  pl.BlockSpec((_BN, 1), lambda i: (i, 0)),
                  pl.BlockSpec((2, _BN, 128), lambda i: (0, i, 0)),
                  pl.BlockSpec((_BN, D), lambda i: (i, 0)),
                  pl.BlockSpec((D, D), lambda i: (0, 0)),
                  pl.BlockSpec((D, D), lambda i: (0, 0)),
                  pl.BlockSpec((16, D), lambda i: (0, 0)),
                  pl.BlockSpec((16, D), lambda i: (0, 0))],
        out_specs=pl.BlockSpec((_BN, D), lambda i: (i, 0)),
        out_shape=jax.ShapeDtypeStruct((N, D), jnp.float32),
    )(g02, eap, dinv2, hs2, gn, wmt, wst, mm, ms)


def kernel(x, edge_index, edge_attr, gaussian_noise,
           W_mean_lin, b_mean_lin, W_mean_edge, b_mean_edge,
           W_std_lin, b_std_lin, W_std_edge, b_std_edge, prelu_a):
    N, D = x.shape
    E = edge_index.shape[1]
    f32 = jnp.float32

    row = edge_index[0].astype(jnp.int32)
    col = edge_index[1].astype(jnp.int32)
    NP = -(-N // 128) * 128
    rpt = NP // _NT
    ones_b = jnp.ones((_C, 16), f32)
    zeros_e = jnp.zeros((rpt, 16), f32)
    zeros_m = jnp.zeros((rpt, 128), f32)

    nch = E // _C
    cpt = -(-nch // (_NT * 8)) * 8
    padc = _NT * cpt - nch
    row2d = jnp.concatenate([row.reshape(nch, _C),
                             jnp.zeros((padc, _C), jnp.int32)])
    col2d = jnp.concatenate([col.reshape(nch, _C),
                             jnp.zeros((padc, _C), jnp.int32)])
    row23 = jnp.stack([row2d, row2d + N])
    ea_flat = jnp.concatenate(
        [jnp.concatenate([edge_attr, jnp.ones((E, 1), f32),
                          jnp.zeros((E, 6), f32)], axis=1).reshape(-1),
         jnp.zeros((padc * _C * 16,), f32)])

    deg_part = _sc_hist(row2d, ones_b, zeros_e, NP, E)
    hs2, dinv2 = _tc_prep(prelu_a, x, deg_part)
    hs_cat = hs2.reshape(2 * N, 128)
    dinv_flat = dinv2.reshape(N)
    g02 = _sc_main(hs_cat, row23, col2d, zeros_m, N, NP, E)
    eap = _sc_ea(row2d, col2d, ea_flat, dinv_flat, zeros_e, N, NP, E)

    def m_of(W_edge, b_lin, b_edge):
        m = jnp.zeros((16, D), f32)
        m = m.at[:9, :].set(W_edge.T)
        return m.at[9, :].set(b_lin + b_edge)

    out = _tc_final(g02, eap, dinv2, hs2, gaussian_noise,
                    W_mean_lin.T, W_std_lin.T,
                    m_of(W_mean_edge, b_mean_lin, b_mean_edge),
                    m_of(W_std_edge, b_std_lin, b_std_edge))
    return out


# speedup vs baseline: 1.0416x; 1.0416x over previous
import functools

import jax
import jax.numpy as jnp
from jax import lax
from jax.experimental import pallas as pl
from jax.experimental.pallas import tpu as pltpu
from jax.experimental.pallas import tpu_sc as plsc

_C = 128
_NT = 16
_BN = 1000
G_PAD = 8


def _sc_hist(row2d, ones_b, zeros_e, NP, E):
    nch = E // _C
    halfc = -(-nch // (2 * _NT)) * _NT
    cpt = halfc // _NT
    rpt = NP // _NT
    mesh = plsc.VectorSubcoreMesh(core_axis_name="c", subcore_axis_name="s")

    @functools.partial(
        pl.kernel, mesh=mesh,
        out_type=jax.ShapeDtypeStruct((2, NP, 16), jnp.float32),
        scratch_types=[
            pltpu.VMEM((cpt, _C), jnp.int32),
            pltpu.VMEM((_C, 16), jnp.float32),
            pltpu.VMEM_SHARED((NP, 16), jnp.float32),
        ],
        compiler_params=pltpu.CompilerParams(use_tc_tiling_on_sc=False),
    )
    def hist(row2d_hbm, ones_hbm, zeros_hbm, out_hbm, ridx_all, ones_v,
             acc_sh):
        c = lax.axis_index("c")
        s = lax.axis_index("s")
        off = c * halfc + s * cpt
        count = jnp.clip(nch - off, 0, cpt)
        pltpu.sync_copy(zeros_hbm, acc_sh.at[pl.ds(s * rpt, rpt)])
        pltpu.sync_copy(ones_hbm, ones_v)
        pltpu.sync_copy(row2d_hbm.at[pl.ds(off, cpt)], ridx_all)
        plsc.subcore_barrier()

        def body(j, carry):
            @pl.when(j < count)
            def _():
                pltpu.sync_copy(ones_v, acc_sh.at[ridx_all.at[j]], add=True)

            return carry

        lax.fori_loop(0, cpt, body, 0)
        plsc.subcore_barrier()
        pltpu.sync_copy(acc_sh.at[pl.ds(s * rpt, rpt)],
                        out_hbm.at[c, pl.ds(s * rpt, rpt)])

    return hist(row2d, ones_b, zeros_e)


def _sc_main(hs_cat, row23, col2d, zeros_m, N, NP, E):
    nch = E // _C
    cpt = -(-nch // (_NT * 8)) * 8
    G = 8
    ngrp = cpt // G
    rpt = NP // _NT
    mesh = plsc.VectorSubcoreMesh(core_axis_name="c", subcore_axis_name="s")

    @functools.partial(
        pl.kernel, mesh=mesh,
        out_type=jax.ShapeDtypeStruct((2, NP, 128), jnp.float32),
        scratch_types=[
            pltpu.VMEM((2, G, _C), jnp.int32),
            pltpu.VMEM((2, G, _C), jnp.int32),
            pltpu.VMEM((2, _C, 128), jnp.float32),
            pltpu.VMEM_SHARED((NP, 128), jnp.float32),
            pltpu.SemaphoreType.DMA,
            pltpu.SemaphoreType.DMA,
        ],
    )
    def main(hs_hbm, row23_hbm, col2d_hbm, zm_hbm,
             g0_hbm, ridx2, cidx2, gbuf, accm, sem0, sem1):
        c = lax.axis_index("c")
        s = lax.axis_index("s")
        off = s * cpt
        count = jnp.clip(nch - off, 0, cpt)
        pltpu.sync_copy(zm_hbm, accm.at[pl.ds(s * rpt, rpt)])
        plsc.subcore_barrier()
        sems = [sem0, sem1]

        def load_group(g):
            p = g % 2
            pltpu.sync_copy(row23_hbm.at[c, pl.ds(off + g * G, G)],
                            ridx2.at[p])
            pltpu.sync_copy(col2d_hbm.at[pl.ds(off + g * G, G)], cidx2.at[p])

        def scatter_chunk(k):
            g, j = k // G, k % G

            @pl.when(k < count)
            def _():
                pltpu.make_async_copy(hs_hbm.at[ridx2.at[g % 2, j]],
                                      gbuf.at[k % 2], sems[k % 2]).wait()
                pltpu.sync_copy(gbuf.at[k % 2],
                                accm.at[cidx2.at[g % 2, j]], add=True)

        load_group(0)
        for k in range(cpt):
            g, j = k // G, k % G

            @pl.when(k < count)
            def _(g=g, j=j, k=k):
                pltpu.async_copy(hs_hbm.at[ridx2.at[g % 2, j]],
                                 gbuf.at[k % 2], sems[k % 2])

            if k >= 1:
                scatter_chunk(k - 1)
            if j == 0 and g + 1 < ngrp:
                load_group(g + 1)
        scatter_chunk(cpt - 1)

        plsc.subcore_barrier()
        pltpu.sync_copy(accm.at[pl.ds(s * rpt, rpt)],
                        g0_hbm.at[c, pl.ds(s * rpt, rpt)])

    return main(hs_cat, row23, col2d, zeros_m)


def _sc_ea(row2d, col2d, ea_flat, dinv, zeros_e, N, NP, E):
    nch = E // _C
    halfc = -(-nch // (2 * _NT)) * _NT
    cpt = halfc // _NT
    rpt = NP // _NT
    mesh = plsc.VectorSubcoreMesh(core_axis_name="c", subcore_axis_name="s")

    @functools.partial(
        pl.kernel, mesh=mesh,
        out_type=jax.ShapeDtypeStruct((2, NP, 16), jnp.float32),
        scratch_types=[
            pltpu.VMEM((cpt, _C), jnp.int32),
            pltpu.VMEM((cpt, _C), jnp.int32),
            pltpu.VMEM((cpt * _C * 16,), jnp.float32),
            pltpu.VMEM((_C, 16), jnp.float32),
            pltpu.VMEM((N,), jnp.float32),
            pltpu.VMEM_SHARED((NP, 16), jnp.float32),
        ],
        compiler_params=pltpu.CompilerParams(needs_layout_passes=False,
                                             use_tc_tiling_on_sc=False),
    )
    def eak(row2d_hbm, col2d_hbm, ea_hbm, dinv_hbm, ze_hbm,
            eap_hbm, ridx_all, cidx_all, ea_all, ea2, dinv_v, acce):
        c = lax.axis_index("c")
        s = lax.axis_index("s")
        off = c * halfc + s * cpt
        count = jnp.clip(nch - off, 0, cpt)
        pltpu.sync_copy(ze_hbm, acce.at[pl.ds(s * rpt, rpt)])
        pltpu.sync_copy(dinv_hbm, dinv_v)
        pltpu.sync_copy(row2d_hbm.at[pl.ds(off, cpt)], ridx_all)
        pltpu.sync_copy(col2d_hbm.at[pl.ds(off, cpt)], cidx_all)
        pltpu.sync_copy(ea_hbm.at[pl.ds(off * _C * 16, cpt * _C * 16)], ea_all)
        plsc.subcore_barrier()

        def ebody(j, carry):
            @pl.when(j < count)
            def _():
                ebase = j * (_C * 16)
                for q in range(_C // 16):
                    r16 = ridx_all[j, pl.ds(q * 16, 16)]
                    g16 = plsc.load_gather(dinv_v, [r16])
                    e16 = lax.broadcasted_iota(jnp.int32, (16,), 0) + (q * 16)
                    for ch in range(16):
                        chv = jnp.full((16,), ch, jnp.int32)
                        v = plsc.load_gather(ea_all, [ebase + e16 * 16 + chv])
                        plsc.store_scatter(ea2, [e16, chv], v * g16)
                pltpu.sync_copy(ea2, acce.at[cidx_all.at[j]], add=True)

            return carry

        lax.fori_loop(0, cpt, ebody, 0)
        plsc.subcore_barrier()
        pltpu.sync_copy(acce.at[pl.ds(s * rpt, rpt)],
                        eap_hbm.at[c, pl.ds(s * rpt, rpt)])

    return eak(row2d, col2d, ea_flat, dinv, zeros_e)


def _prep_body(a_ref, x_ref, dp_ref, hs_ref, dinv_ref):
    a = a_ref[0]
    x = x_ref[...]
    dp = dp_ref[...]
    cnt = dp[0, :, 0:1] + dp[1, :, 0:1]
    dinv = lax.rsqrt(cnt + 1.0)
    hidden = jnp.where(x >= 0, x, a * x)
    hs = hidden * dinv
    dinv_ref[...] = dinv
    hs_ref[0] = hs[:, :128]
    hs_ref[1] = hs[:, 128:]


def _tc_prep(prelu_a, x, deg_part):
    N, D = x.shape
    grid = (N // _BN,)
    return pl.pallas_call(
        _prep_body,
        grid=grid,
        in_specs=[pl.BlockSpec(memory_space=pltpu.SMEM),
                  pl.BlockSpec((_BN, D), lambda i: (i, 0)),
                  pl.BlockSpec((2, _BN, 16), lambda i: (0, i, 0))],
        out_specs=[pl.BlockSpec((2, _BN, 128), lambda i: (0, i, 0)),
                   pl.BlockSpec((_BN, 1), lambda i: (i, 0))],
        out_shape=[jax.ShapeDtypeStruct((2, N, 128), jnp.float32),
                   jax.ShapeDtypeStruct((N, 1), jnp.float32)],
    )(jnp.reshape(prelu_a, (1,)), x, deg_part)


def _final_body(g02_ref, eap_ref, dinv_ref, hs2_ref, gn_ref,
                wmt_ref, wst_ref, mm_ref, ms_ref, out_ref):
    dinv = dinv_ref[...]
    g0 = jnp.concatenate([g02_ref[0], g02_ref[1]], axis=1)
    hs = jnp.concatenate([hs2_ref[0], hs2_ref[1]], axis=1)
    ea = eap_ref[0] + eap_ref[1]
    ch = lax.broadcasted_iota(jnp.int32, (1, 16), 1)
    sl = ((ch == 7) | (ch == 9)).astype(jnp.float32)
    a16 = (ea + dinv * sl) * dinv
    h_full = (g0 + hs) * dinv
    mean = (jnp.dot(h_full, wmt_ref[...], preferred_element_type=jnp.float32)
            + jnp.dot(a16, mm_ref[...], preferred_element_type=jnp.float32))
    logstd = (jnp.dot(h_full, wst_ref[...], preferred_element_type=jnp.float32)
              + jnp.dot(a16, ms_ref[...], preferred_element_type=jnp.float32))
    out_ref[...] = gn_ref[...] * jnp.exp(logstd) + mean


def _tc_final(g02, eap, dinv2, hs2, gn, wmt, wst, mm, ms):
    N, D = gn.shape
    grid = (N // _BN,)
    return pl.pallas_call(
        _final_body,
        grid=grid,
        in_specs=[pl.BlockSpec((2, _BN, 128), lambda i: (0, i, 0)),
                  pl.BlockSpec((2, _BN, 16), lambda i: (0, i, 0)),
                  pl.BlockSpec((_BN, 1), lambda i: (i, 0)),
                  pl.BlockSpec((2, _BN, 128), lambda i: (0, i, 0)),
                  pl.BlockSpec((_BN, D), lambda i: (i, 0)),
                  pl.BlockSpec((D, D), lambda i: (0, 0)),
                  pl.BlockSpec((D, D), lambda i: (0, 0)),
                  pl.BlockSpec((16, D), lambda i: (0, 0)),
                  pl.BlockSpec((16, D), lambda i: (0, 0))],
        out_specs=pl.BlockSpec((_BN, D), lambda i: (i, 0)),
        out_shape=jax.ShapeDtypeStruct((N, D), jnp.float32),
    )(g02, eap, dinv2, hs2, gn, wmt, wst, mm, ms)


def kernel(x, edge_index, edge_attr, gaussian_noise,
           W_mean_lin, b_mean_lin, W_mean_edge, b_mean_edge,
           W_std_lin, b_std_lin, W_std_edge, b_std_edge, prelu_a):
    N, D = x.shape
    E = edge_index.shape[1]
    f32 = jnp.float32

    row = edge_index[0].astype(jnp.int32)
    col = edge_index[1].astype(jnp.int32)
    NP = -(-N // 128) * 128
    rpt = NP // _NT
    ones_b = jnp.ones((_C, 16), f32)
    zeros_e = jnp.zeros((rpt, 16), f32)
    zeros_m = jnp.zeros((rpt, 128), f32)

    nch = E // _C
    cpt = -(-nch // (_NT * 8)) * 8
    padc = _NT * cpt - nch
    row2d = jnp.concatenate([row.reshape(nch, _C),
                             jnp.zeros((padc, _C), jnp.int32)])
    col2d = jnp.concatenate([col.reshape(nch, _C),
                             jnp.zeros((padc, _C), jnp.int32)])
    row23 = jnp.stack([row2d, row2d + N])
    ea_flat = jnp.concatenate(
        [jnp.concatenate([edge_attr, jnp.ones((E, 1), f32),
                          jnp.zeros((E, 6), f32)], axis=1).reshape(-1),
         jnp.zeros((padc * _C * 16,), f32)])

    deg_part = _sc_hist(row2d, ones_b, zeros_e, NP, E)
    hs2, dinv2 = _tc_prep(prelu_a, x, deg_part)
    hs_cat = hs2.reshape(2 * N, 128)
    dinv_flat = dinv2.reshape(N)
    g02 = _sc_main(hs_cat, row23, col2d, zeros_m, N, NP, E)
    eap = _sc_ea(row2d, col2d, ea_flat, dinv_flat, zeros_e, N, NP, E)

    def m_of(W_edge, b_lin, b_edge):
        m = jnp.zeros((16, D), f32)
        m = m.at[:9, :].set(W_edge.T)
        return m.at[9, :].set(b_lin + b_edge)

    out = _tc_final(g02, eap, dinv2, hs2, gaussian_noise,
                    W_mean_lin.T, W_std_lin.T,
                    m_of(W_mean_edge, b_mean_lin, b_mean_edge),
                    m_of(W_std_edge, b_std_lin, b_std_edge))
    return out


# speedup vs baseline: 1.0598x; 1.0175x over previous
import functools

import jax
import jax.numpy as jnp
from jax import lax
from jax.experimental import pallas as pl
from jax.experimental.pallas import tpu as pltpu
from jax.experimental.pallas import tpu_sc as plsc

_C = 128
_NT = 16
_BN = 1000
G_PAD = 8


def _sc_hist(row2d, ones_b, zeros_e, NP, E):
    nch = E // _C
    halfc = -(-nch // (2 * _NT)) * _NT
    cpt = halfc // _NT
    rpt = NP // _NT
    mesh = plsc.VectorSubcoreMesh(core_axis_name="c", subcore_axis_name="s")

    @functools.partial(
        pl.kernel, mesh=mesh,
        out_type=jax.ShapeDtypeStruct((2, NP, 16), jnp.float32),
        scratch_types=[
            pltpu.VMEM((cpt, _C), jnp.int32),
            pltpu.VMEM((_C, 16), jnp.float32),
            pltpu.VMEM_SHARED((NP, 16), jnp.float32),
        ],
        compiler_params=pltpu.CompilerParams(use_tc_tiling_on_sc=False),
    )
    def hist(row2d_hbm, ones_hbm, zeros_hbm, out_hbm, ridx_all, ones_v,
             acc_sh):
        c = lax.axis_index("c")
        s = lax.axis_index("s")
        off = c * halfc + s * cpt
        count = jnp.clip(nch - off, 0, cpt)
        pltpu.sync_copy(zeros_hbm, acc_sh.at[pl.ds(s * rpt, rpt)])
        pltpu.sync_copy(ones_hbm, ones_v)
        pltpu.sync_copy(row2d_hbm.at[pl.ds(off, cpt)], ridx_all)
        plsc.subcore_barrier()

        def body(j, carry):
            @pl.when(j < count)
            def _():
                pltpu.sync_copy(ones_v, acc_sh.at[ridx_all.at[j]], add=True)

            return carry

        lax.fori_loop(0, cpt, body, 0)
        plsc.subcore_barrier()
        pltpu.sync_copy(acc_sh.at[pl.ds(s * rpt, rpt)],
                        out_hbm.at[c, pl.ds(s * rpt, rpt)])

    return hist(row2d, ones_b, zeros_e)


def _sc_main(hs_cat, row23, col2d, zeros_m, N, NP, E):
    nch = E // _C
    cpt = -(-nch // (_NT * 8)) * 8
    G = 8
    ngrp = cpt // G
    rpt = NP // _NT
    mesh = plsc.VectorSubcoreMesh(core_axis_name="c", subcore_axis_name="s")

    @functools.partial(
        pl.kernel, mesh=mesh,
        out_type=jax.ShapeDtypeStruct((2, NP, 128), jnp.float32),
        scratch_types=[
            pltpu.VMEM((2, G, _C), jnp.int32),
            pltpu.VMEM((2, G, _C), jnp.int32),
            pltpu.VMEM((2, _C, 128), jnp.float32),
            pltpu.VMEM_SHARED((NP, 128), jnp.float32),
            pltpu.SemaphoreType.DMA,
            pltpu.SemaphoreType.DMA,
        ],
    )
    def main(hs_hbm, row23_hbm, col2d_hbm, zm_hbm,
             g0_hbm, ridx2, cidx2, gbuf, accm, sem0, sem1):
        c = lax.axis_index("c")
        s = lax.axis_index("s")
        off = s * cpt
        count = jnp.clip(nch - off, 0, cpt)
        pltpu.sync_copy(zm_hbm, accm.at[pl.ds(s * rpt, rpt)])
        plsc.subcore_barrier()
        sems = [sem0, sem1]

        def load_group(g):
            p = g % 2
            pltpu.sync_copy(row23_hbm.at[c, pl.ds(off + g * G, G)],
                            ridx2.at[p])
            pltpu.sync_copy(col2d_hbm.at[pl.ds(off + g * G, G)], cidx2.at[p])

        def scatter_chunk(k):
            g, j = k // G, k % G

            @pl.when(k < count)
            def _():
                pltpu.make_async_copy(hs_hbm.at[ridx2.at[g % 2, j]],
                                      gbuf.at[k % 2], sems[k % 2]).wait()
                pltpu.sync_copy(gbuf.at[k % 2],
                                accm.at[cidx2.at[g % 2, j]], add=True)

        load_group(0)
        for k in range(cpt):
            g, j = k // G, k % G

            @pl.when(k < count)
            def _(g=g, j=j, k=k):
                pltpu.async_copy(hs_hbm.at[ridx2.at[g % 2, j]],
                                 gbuf.at[k % 2], sems[k % 2])

            if k >= 1:
                scatter_chunk(k - 1)
            if j == 0 and g + 1 < ngrp:
                load_group(g + 1)
        scatter_chunk(cpt - 1)

        plsc.subcore_barrier()
        pltpu.sync_copy(accm.at[pl.ds(s * rpt, rpt)],
                        g0_hbm.at[c, pl.ds(s * rpt, rpt)])

    return main(hs_cat, row23, col2d, zeros_m)


def _sc_ea(row2d, col2d, ea_flat, dinv, zeros_e, N, NP, E):
    nch = E // _C
    halfc = -(-nch // (2 * _NT)) * _NT
    cpt = halfc // _NT
    rpt = NP // _NT
    mesh = plsc.VectorSubcoreMesh(core_axis_name="c", subcore_axis_name="s")

    @functools.partial(
        pl.kernel, mesh=mesh,
        out_type=jax.ShapeDtypeStruct((2, NP, 16), jnp.float32),
        scratch_types=[
            pltpu.VMEM((cpt, _C), jnp.int32),
            pltpu.VMEM((cpt, _C), jnp.int32),
            pltpu.VMEM((cpt * _C * 16,), jnp.float32),
            pltpu.VMEM((2, _C, 16), jnp.float32),
            pltpu.VMEM((N,), jnp.float32),
            pltpu.VMEM_SHARED((NP, 16), jnp.float32),
            pltpu.SemaphoreType.DMA,
        ],
        compiler_params=pltpu.CompilerParams(needs_layout_passes=False,
                                             use_tc_tiling_on_sc=False),
    )
    def eak(row2d_hbm, col2d_hbm, ea_hbm, dinv_hbm, ze_hbm,
            eap_hbm, ridx_all, cidx_all, ea_all, ea2, dinv_v, acce, sem):
        c = lax.axis_index("c")
        s = lax.axis_index("s")
        off = c * halfc + s * cpt
        count = jnp.clip(nch - off, 0, cpt)
        pltpu.sync_copy(ze_hbm, acce.at[pl.ds(s * rpt, rpt)])
        pltpu.sync_copy(dinv_hbm, dinv_v)
        pltpu.sync_copy(row2d_hbm.at[pl.ds(off, cpt)], ridx_all)
        pltpu.sync_copy(col2d_hbm.at[pl.ds(off, cpt)], cidx_all)
        pltpu.sync_copy(ea_hbm.at[pl.ds(off * _C * 16, cpt * _C * 16)], ea_all)
        plsc.subcore_barrier()

        def ebody(j, carry):
            @pl.when(j < count)
            def _():
                b = j % 2
                bv = jnp.full((16,), b, jnp.int32)
                ebase = j * (_C * 16)
                for q in range(_C // 16):
                    r16 = ridx_all[j, pl.ds(q * 16, 16)]
                    g16 = plsc.load_gather(dinv_v, [r16])
                    e16 = lax.broadcasted_iota(jnp.int32, (16,), 0) + (q * 16)
                    for ch in range(16):
                        chv = jnp.full((16,), ch, jnp.int32)
                        v = plsc.load_gather(ea_all, [ebase + e16 * 16 + chv])
                        plsc.store_scatter(ea2, [bv, e16, chv], v * g16)

                @pl.when(j > 0)
                def _():
                    pltpu.make_async_copy(ea2.at[1 - b],
                                          acce.at[cidx_all.at[j - 1]],
                                          sem).wait()

                pltpu.async_copy(ea2.at[b], acce.at[cidx_all.at[j]], sem,
                                 add=True)

            return carry

        lax.fori_loop(0, cpt, ebody, 0)

        @pl.when(count > 0)
        def _():
            pltpu.make_async_copy(ea2.at[(count - 1) % 2],
                                  acce.at[cidx_all.at[count - 1]], sem).wait()
        plsc.subcore_barrier()
        pltpu.sync_copy(acce.at[pl.ds(s * rpt, rpt)],
                        eap_hbm.at[c, pl.ds(s * rpt, rpt)])

    return eak(row2d, col2d, ea_flat, dinv, zeros_e)


def _prep_body(a_ref, x_ref, dp_ref, hs_ref, dinv_ref):
    a = a_ref[0]
    x = x_ref[...]
    dp = dp_ref[...]
    cnt = dp[0, :, 0:1] + dp[1, :, 0:1]
    dinv = lax.rsqrt(cnt + 1.0)
    hidden = jnp.where(x >= 0, x, a * x)
    hs = hidden * dinv
    dinv_ref[...] = dinv
    hs_ref[0] = hs[:, :128]
    hs_ref[1] = hs[:, 128:]


def _tc_prep(prelu_a, x, deg_part):
    N, D = x.shape
    grid = (N // _BN,)
    return pl.pallas_call(
        _prep_body,
        grid=grid,
        in_specs=[pl.BlockSpec(memory_space=pltpu.SMEM),
                  pl.BlockSpec((_BN, D), lambda i: (i, 0)),
                  pl.BlockSpec((2, _BN, 16), lambda i: (0, i, 0))],
        out_specs=[pl.BlockSpec((2, _BN, 128), lambda i: (0, i, 0)),
                   pl.BlockSpec((_BN, 1), lambda i: (i, 0))],
        out_shape=[jax.ShapeDtypeStruct((2, N, 128), jnp.float32),
                   jax.ShapeDtypeStruct((N, 1), jnp.float32)],
    )(jnp.reshape(prelu_a, (1,)), x, deg_part)


def _final_body(g02_ref, eap_ref, dinv_ref, hs2_ref, gn_ref,
                wmt_ref, wst_ref, mm_ref, ms_ref, out_ref):
    dinv = dinv_ref[...]
    g0 = jnp.concatenate([g02_ref[0], g02_ref[1]], axis=1)
    hs = jnp.concatenate([hs2_ref[0], hs2_ref[1]], axis=1)
    ea = eap_ref[0] + eap_ref[1]
    ch = lax.broadcasted_iota(jnp.int32, (1, 16), 1)
    sl = ((ch == 7) | (ch == 9)).astype(jnp.float32)
    a16 = (ea + dinv * sl) * dinv
    h_full = (g0 + hs) * dinv
    mean = (jnp.dot(h_full, wmt_ref[...], preferred_element_type=jnp.float32)
            + jnp.dot(a16, mm_ref[...], preferred_element_type=jnp.float32))
    logstd = (jnp.dot(h_full, wst_ref[...], preferred_element_type=jnp.float32)
              + jnp.dot(a16, ms_ref[...], preferred_element_type=jnp.float32))
    out_ref[...] = gn_ref[...] * jnp.exp(logstd) + mean


def _tc_final(g02, eap, dinv2, hs2, gn, wmt, wst, mm, ms):
    N, D = gn.shape
    grid = (N // _BN,)
    return pl.pallas_call(
        _final_body,
        grid=grid,
        in_specs=[pl.BlockSpec((2, _BN, 128), lambda i: (0, i, 0)),
                  pl.BlockSpec((2, _BN, 16), lambda i: (0, i, 0)),
                  pl.BlockSpec((_BN, 1), lambda i: (i, 0)),
                  pl.BlockSpec((2, _BN, 128), lambda i: (0, i, 0)),
                  pl.BlockSpec((_BN, D), lambda i: (i, 0)),
                  pl.BlockSpec((D, D), lambda i: (0, 0)),
                  pl.BlockSpec((D, D), lambda i: (0, 0)),
                  pl.BlockSpec((16, D), lambda i: (0, 0)),
                  pl.BlockSpec((16, D), lambda i: (0, 0))],
        out_specs=pl.BlockSpec((_BN, D), lambda i: (i, 0)),
        out_shape=jax.ShapeDtypeStruct((N, D), jnp.float32),
    )(g02, eap, dinv2, hs2, gn, wmt, wst, mm, ms)


def kernel(x, edge_index, edge_attr, gaussian_noise,
           W_mean_lin, b_mean_lin, W_mean_edge, b_mean_edge,
           W_std_lin, b_std_lin, W_std_edge, b_std_edge, prelu_a):
    N, D = x.shape
    E = edge_index.shape[1]
    f32 = jnp.float32

    row = edge_index[0].astype(jnp.int32)
    col = edge_index[1].astype(jnp.int32)
    NP = -(-N // 128) * 128
    rpt = NP // _NT
    ones_b = jnp.ones((_C, 16), f32)
    zeros_e = jnp.zeros((rpt, 16), f32)
    zeros_m = jnp.zeros((rpt, 128), f32)

    nch = E // _C
    cpt = -(-nch // (_NT * 8)) * 8
    padc = _NT * cpt - nch
    row2d = jnp.concatenate([row.reshape(nch, _C),
                             jnp.zeros((padc, _C), jnp.int32)])
    col2d = jnp.concatenate([col.reshape(nch, _C),
                             jnp.zeros((padc, _C), jnp.int32)])
    row23 = jnp.stack([row2d, row2d + N])
    ea_flat = jnp.concatenate(
        [jnp.concatenate([edge_attr, jnp.ones((E, 1), f32),
                          jnp.zeros((E, 6), f32)], axis=1).reshape(-1),
         jnp.zeros((padc * _C * 16,), f32)])

    deg_part = _sc_hist(row2d, ones_b, zeros_e, NP, E)
    hs2, dinv2 = _tc_prep(prelu_a, x, deg_part)
    hs_cat = hs2.reshape(2 * N, 128)
    dinv_flat = dinv2.reshape(N)
    g02 = _sc_main(hs_cat, row23, col2d, zeros_m, N, NP, E)
    eap = _sc_ea(row2d, col2d, ea_flat, dinv_flat, zeros_e, N, NP, E)

    def m_of(W_edge, b_lin, b_edge):
        m = jnp.zeros((16, D), f32)
        m = m.at[:9, :].set(W_edge.T)
        return m.at[9, :].set(b_lin + b_edge)

    out = _tc_final(g02, eap, dinv2, hs2, gaussian_noise,
                    W_mean_lin.T, W_std_lin.T,
                    m_of(W_mean_edge, b_mean_lin, b_mean_edge),
                    m_of(W_std_edge, b_std_lin, b_std_edge))
    return out
